# baseline (device time: 8726 ns/iter reference)
import jax
import jax.numpy as jnp
from jax import lax
from jax.experimental import pallas as pl
from jax.experimental.pallas import tpu as pltpu

N_DEV = 4
N_CHUNK = 2


def kernel(x):
    m_per, n_total = x.shape
    n_per = n_total // N_DEV
    m_chunk = m_per // N_CHUNK

    def body(
        x_ref, out_ref,
        xi8_ref, ri8_ref, sscale_ref, rscale_ref,
        send_sems, recv_sems, ssend_sems, srecv_sems, ready_sems,
    ):
        my_i = lax.axis_index("i")

        barrier_sem = pltpu.get_barrier_semaphore()
        pl.semaphore_signal(barrier_sem, inc=1)
        pl.semaphore_wait(barrier_sem, 1)

        for j in range(1, N_DEV):
            peer = (my_i - j) % N_DEV
            pl.semaphore_signal(
                ready_sems.at[j - 1], inc=1,
                device_id=(peer,), device_id_type=pl.DeviceIdType.MESH,
            )

        def chunk_rdmas(k, c, dst):
            srdma = pltpu.make_async_remote_copy(
                src_ref=sscale_ref.at[k - 1, c],
                dst_ref=rscale_ref.at[k - 1, c],
                send_sem=ssend_sems.at[k - 1, c],
                recv_sem=srecv_sems.at[k - 1, c],
                device_id=(dst,),
                device_id_type=pl.DeviceIdType.MESH,
            )
            rdma = pltpu.make_async_remote_copy(
                src_ref=xi8_ref.at[k - 1, pl.ds(c * m_chunk, m_chunk), :],
                dst_ref=ri8_ref.at[k - 1, pl.ds(c * m_chunk, m_chunk), :],
                send_sem=send_sems.at[k - 1, c],
                recv_sem=recv_sems.at[k - 1, c],
                device_id=(dst,),
                device_id_type=pl.DeviceIdType.MESH,
            )
            return srdma, rdma

        sends = []
        for k in [2, 1, 3]:
            dst = (my_i + k) % N_DEV
            waited = False
            for c in range(N_CHUNK):
                blk = x_ref[pl.ds(c * m_chunk, m_chunk), pl.ds(dst * n_per, n_per)]
                m = jnp.max(jnp.abs(blk))
                scale = jnp.where(m > 0, m, 1.0) / 127.0
                sscale_ref[k - 1, c, :, :] = jnp.full((1, 128), scale, jnp.float32)
                xi8_ref[k - 1, pl.ds(c * m_chunk, m_chunk), :] = jnp.round(
                    blk / scale
                ).astype(jnp.int8)
                if not waited:
                    pl.semaphore_wait(ready_sems.at[k - 1], 1)
                    waited = True
                srdma, rdma = chunk_rdmas(k, c, dst)
                srdma.start()
                rdma.start()
                sends.append((srdma, rdma))

        out_ref[pl.ds(my_i * m_per, m_per), :] = x_ref[
            :, pl.ds(my_i * n_per, n_per)
        ].astype(jnp.bfloat16)

        for k in range(1, N_DEV):
            src = (my_i - k) % N_DEV
            for c in range(N_CHUNK):
                srecv, recv = chunk_rdmas(k, c, src)
                srecv.wait_recv()
                recv.wait_recv()
                out_ref[pl.ds(src * m_per + c * m_chunk, m_chunk), :] = (
                    ri8_ref[k - 1, pl.ds(c * m_chunk, m_chunk), :].astype(
                        jnp.float32
                    )
                    * rscale_ref[k - 1, c, 0:1, 0:1]
                ).astype(jnp.bfloat16)

        for srdma, rdma in sends:
            srdma.wait_send()
            rdma.wait_send()


    return pl.pallas_call(
        body,
        out_shape=jax.ShapeDtypeStruct((N_DEV * m_per, n_per), jnp.bfloat16),
        in_specs=[pl.BlockSpec(memory_space=pltpu.VMEM)],
        out_specs=pl.BlockSpec(memory_space=pltpu.VMEM),
        scratch_shapes=[
            pltpu.VMEM((N_DEV - 1, m_per, n_per), jnp.int8),
            pltpu.VMEM((N_DEV - 1, m_per, n_per), jnp.int8),
            pltpu.VMEM((N_DEV - 1, N_CHUNK, 1, 128), jnp.float32),
            pltpu.VMEM((N_DEV - 1, N_CHUNK, 1, 128), jnp.float32),
            pltpu.SemaphoreType.DMA((N_DEV - 1, N_CHUNK)),
            pltpu.SemaphoreType.DMA((N_DEV - 1, N_CHUNK)),
            pltpu.SemaphoreType.DMA((N_DEV - 1, N_CHUNK)),
            pltpu.SemaphoreType.DMA((N_DEV - 1, N_CHUNK)),
            pltpu.SemaphoreType.REGULAR((N_DEV - 1,)),
        ],
        compiler_params=pltpu.CompilerParams(collective_id=0),
    )(x)
